# baseline (speedup 1.0000x reference)
import functools

import jax
import jax.numpy as jnp
import numpy as np
from jax import lax
from jax.experimental import pallas as pl
from jax.experimental.pallas import tpu as pltpu
from jax.experimental.pallas import tpu_sc as plsc

ITERATIONS = 512
LEN_SAMPLE = 4
THRESHOLD = 5.0
N = 65536
MBLK = 16
NCHUNK = 1024

_SEL_NP = np.asarray(
    (jax.random.uniform(jax.random.key(1), (ITERATIONS, LEN_SAMPLE),
                        dtype=jnp.float32) * (N - 1e-08)).astype(jnp.int32)
).reshape(-1)
_IDX2_NP = np.stack([2 * _SEL_NP, 2 * _SEL_NP + 1], axis=-1).reshape(-1)

_NC = 2
_NS = 16
_NW = _NC * _NS
_SEL_W = (ITERATIONS * LEN_SAMPLE) // _NW
_MOD_W = ITERATIONS // _NW


_ELT_W = 2 * _SEL_W


@functools.partial(
    pl.kernel,
    out_type=[
        jax.ShapeDtypeStruct((ITERATIONS * LEN_SAMPLE * 2,), jnp.float32),
        jax.ShapeDtypeStruct((ITERATIONS * LEN_SAMPLE * 2,), jnp.float32),
    ],
    mesh=plsc.VectorSubcoreMesh(core_axis_name="c", subcore_axis_name="s"),
    scratch_types=[
        pltpu.VMEM((_ELT_W,), jnp.int32),
        pltpu.VMEM((_ELT_W,), jnp.float32),
        pltpu.VMEM((_ELT_W,), jnp.float32),
        pltpu.SemaphoreType.DMA,
        pltpu.SemaphoreType.DMA,
    ],
)
def _sample(xf_hbm, yf_hbm, idx2_hbm, xs_hbm, ys_hbm, idx_v, xr, yr, s1, s2):
    wid = lax.axis_index("s") * _NC + lax.axis_index("c")
    base = wid * _ELT_W
    pltpu.sync_copy(idx2_hbm.at[pl.ds(base, _ELT_W)], idx_v)
    cx = pltpu.async_copy(xf_hbm.at[idx_v], xr, s1)
    cy = pltpu.async_copy(yf_hbm.at[idx_v], yr, s2)
    cx.wait()
    cy.wait()
    pltpu.sync_copy(xr, xs_hbm.at[pl.ds(base, _ELT_W)])
    pltpu.sync_copy(yr, ys_hbm.at[pl.ds(base, _ELT_W)])


def _count_kernel(xt_ref, yt_ref, xs_ref, ys_ref, model_out_ref, cnt_out_ref,
                  counts_ref, m_ref):
    m = pl.program_id(0)

    @pl.when(m == 0)
    def _():
        d = ys_ref[...] - xs_ref[...]
        t0s = ((d[:, 0:1] + d[:, 2:3]) + d[:, 4:5]) + d[:, 6:7]
        t1s = ((d[:, 1:2] + d[:, 3:4]) + d[:, 5:6]) + d[:, 7:8]
        m_ref[:, 0:1] = t0s * (1.0 / LEN_SAMPLE)
        m_ref[:, 1:2] = t1s * (1.0 / LEN_SAMPLE)

    t0 = m_ref[pl.ds(m * MBLK, MBLK), 0:1]
    t1 = m_ref[pl.ds(m * MBLK, MBLK), 1:2]

    nchunks = N // NCHUNK
    accs = [jnp.zeros((MBLK, NCHUNK), jnp.int32) for _ in range(4)]
    for j in range(nchunks):
        x0 = xt_ref[0:1, j * NCHUNK:(j + 1) * NCHUNK]
        x1 = xt_ref[1:2, j * NCHUNK:(j + 1) * NCHUNK]
        y0 = yt_ref[0:1, j * NCHUNK:(j + 1) * NCHUNK]
        y1 = yt_ref[1:2, j * NCHUNK:(j + 1) * NCHUNK]
        a = (x0 + t0) - y0
        b = (x1 + t1) - y1
        r = a * a + b * b
        accs[j % 4] = accs[j % 4] + (r < THRESHOLD * THRESHOLD).astype(jnp.int32)
    acc = (accs[0] + accs[1]) + (accs[2] + accs[3])
    counts_ref[pl.ds(m * MBLK, MBLK), :] = jnp.sum(acc, axis=1, keepdims=True)

    @pl.when(m == pl.num_programs(0) - 1)
    def _():
        counts = counts_ref[...]
        maxc = jnp.max(counts)
        ii = jax.lax.broadcasted_iota(jnp.int32, (ITERATIONS, 1), 0)
        best = jnp.min(jnp.where(counts == maxc, ii, ITERATIONS))
        sel = ii == best
        model_out_ref[0] = jnp.sum(jnp.where(sel, m_ref[:, 0:1], 0.0))
        model_out_ref[1] = jnp.sum(jnp.where(sel, m_ref[:, 1:2], 0.0))
        cnt_out_ref[0] = maxc


def _score(xt, yt, xs, ys):
    return pl.pallas_call(
        _count_kernel,
        grid=(ITERATIONS // MBLK,),
        in_specs=[
            pl.BlockSpec((2, N), lambda m: (0, 0)),
            pl.BlockSpec((2, N), lambda m: (0, 0)),
            pl.BlockSpec((ITERATIONS, 2 * LEN_SAMPLE), lambda m: (0, 0)),
            pl.BlockSpec((ITERATIONS, 2 * LEN_SAMPLE), lambda m: (0, 0)),
        ],
        out_specs=[
            pl.BlockSpec(memory_space=pltpu.SMEM),
            pl.BlockSpec(memory_space=pltpu.SMEM),
        ],
        out_shape=[
            jax.ShapeDtypeStruct((2,), jnp.float32),
            jax.ShapeDtypeStruct((1,), jnp.int32),
        ],
        scratch_shapes=[
            pltpu.VMEM((ITERATIONS, 1), jnp.int32),
            pltpu.VMEM((ITERATIONS, 2), jnp.float32),
        ],
    )(xt, yt, xs, ys)


def kernel(x, y):
    xs, ys = _sample(x.reshape(-1), y.reshape(-1), jnp.asarray(_IDX2_NP))
    xs = xs.reshape(ITERATIONS, 2 * LEN_SAMPLE)
    ys = ys.reshape(ITERATIONS, 2 * LEN_SAMPLE)
    model_out, cnt_out = _score(x.T, y.T, xs, ys)
    return (model_out, cnt_out[0])
